# baseline (speedup 1.0000x reference)
import functools

import jax
import jax.numpy as jnp
from jax import lax
from jax.experimental import pallas as pl
from jax.experimental.pallas import tpu as pltpu
from jax.experimental.pallas import tpu_sc as plsc

N_NODES = 10000
N_EDGES = 320000
NP = 10240
NC, NS = 2, 16
NW = NC * NS
EW = N_EDGES // NW
COLS = 80
ROWS = EW // COLS
GROUPS = COLS // 16
NODE_CHUNK = NP // NW
TAIL = N_NODES - (NW - 1) * NODE_CHUNK

_MESH = plsc.VectorSubcoreMesh(
    core_axis_name="c", subcore_axis_name="s", num_cores=NC, num_subcores=NS
)


def _tc_fused(x, Wt, attr8, Wblk, Cdiff):
    E8 = N_EDGES // 8
    EB = E8 // 10

    def body(x_ref, w_ref, a_ref, wb_ref, c_ref, s_ref, p_ref, g_ref):
        @pl.when(pl.program_id(0) == 0)
        def _node_scores():
            s_ref[...] = jnp.sum(x_ref[...] * w_ref[...], axis=1)

        p2 = jnp.dot(a_ref[...], wb_ref[...],
                     preferred_element_type=jnp.float32)
        p_ref[...] = p2
        d = jnp.dot(p2, c_ref[...], preferred_element_type=jnp.float32)
        g_ref[...] = 1.0 / (1.0 + jnp.exp(-d))

    return pl.pallas_call(
        body,
        grid=(10,),
        in_specs=[
            pl.BlockSpec((N_NODES, 128), lambda i: (0, 0)),
            pl.BlockSpec((1, 128), lambda i: (0, 0)),
            pl.BlockSpec((EB, 128), lambda i: (i, 0)),
            pl.BlockSpec((128, 16), lambda i: (0, 0)),
            pl.BlockSpec((16, 8), lambda i: (0, 0)),
        ],
        out_specs=[
            pl.BlockSpec((N_NODES,), lambda i: (0,)),
            pl.BlockSpec((EB, 16), lambda i: (i, 0)),
            pl.BlockSpec((EB, 8), lambda i: (i, 0)),
        ],
        out_shape=[
            jax.ShapeDtypeStruct((N_NODES,), jnp.float32),
            jax.ShapeDtypeStruct((E8, 16), jnp.float32),
            jax.ShapeDtypeStruct((E8, 8), jnp.float32),
        ],
    )(x, Wt, attr8, Wblk, Cdiff)


@functools.partial(
    pl.kernel,
    out_type=[
        jax.ShapeDtypeStruct((N_EDGES,), jnp.float32),
        jax.ShapeDtypeStruct((NC * NP,), jnp.float32),
        jax.ShapeDtypeStruct((NC * NP,), jnp.float32),
    ],
    mesh=_MESH,
    compiler_params=pltpu.CompilerParams(needs_layout_passes=False),
    scratch_types=[
        pltpu.VMEM((N_NODES,), jnp.float32),
        pltpu.VMEM((EW,), jnp.int32),
        pltpu.VMEM((EW,), jnp.int32),
        pltpu.VMEM((ROWS, COLS), jnp.int32),
        pltpu.VMEM((EW,), jnp.float32),
        pltpu.VMEM((16,), jnp.float32),
        pltpu.VMEM((ROWS, COLS), jnp.float32),
        pltpu.VMEM((ROWS, COLS), jnp.float32),
        pltpu.VMEM((EW,), jnp.float32),
        pltpu.VMEM((NP // NS,), jnp.float32),
        pltpu.VMEM_SHARED((NP,), jnp.float32),
        pltpu.VMEM_SHARED((NP,), jnp.float32),
        pltpu.SemaphoreType.DMA,
        pltpu.SemaphoreType.DMA,
    ],
)
def _sc_edge_kernel(s_hbm, src_hbm, dst_hbm, g0_hbm, nw_hbm,
                    ex_hbm, pd_hbm, pn_hbm,
                    s_v, src_v, dst_v, dst2_v, g0_v, nw_v, ex_v, exsi_v,
                    exf_v, z_v, acc_d, acc_n, sem_d, sem_n):
    c = lax.axis_index("c")
    s = lax.axis_index("s")
    wid = s * NC + c

    zchunk = NP // NS

    def zero_body(i, carry):
        z_v[pl.ds(i * 16, 16)] = jnp.zeros((16,), jnp.float32)
        return carry

    lax.fori_loop(0, zchunk // 16, zero_body, 0)
    pltpu.sync_copy(z_v, acc_d.at[pl.ds(s * zchunk, zchunk)])
    pltpu.sync_copy(z_v, acc_n.at[pl.ds(s * zchunk, zchunk)])

    base = wid * EW
    pltpu.sync_copy(s_hbm, s_v)
    pltpu.sync_copy(src_hbm.at[pl.ds(base, EW)], src_v)
    pltpu.sync_copy(dst_hbm.at[pl.ds(base, EW)], dst_v)
    pltpu.sync_copy(g0_hbm.at[pl.ds(base, EW)], g0_v)
    pltpu.sync_copy(nw_hbm, nw_v)

    plsc.subcore_barrier()

    nw = nw_v[pl.ds(0, 16)]
    w00 = nw[0]
    w01 = nw[1]
    w10 = nw[2]
    w11 = nw[3]

    def row_body(r, carry):
        for j in range(GROUPS):
            sl = pl.ds(r * COLS + j * 16, 16)
            csl = pl.ds(j * 16, 16)
            src16 = src_v[sl]
            dst16 = dst_v[sl]
            dst2_v[r, csl] = dst16
            g0 = g0_v[sl]
            si = plsc.load_gather(s_v, [src16])
            sj = plsc.load_gather(s_v, [dst16])
            a0 = si * w00 + sj * w10
            a1 = si * w01 + sj * w11
            a0 = jnp.where(a0 >= 0.0, a0, a0 * 0.2)
            a1 = jnp.where(a1 >= 0.0, a1, a1 * 0.2)
            logit = a0 * g0 + a1 * (1.0 - g0)
            ex = jnp.exp(logit)
            ex_v[r, csl] = ex
            exf_v[sl] = ex
            exsi_v[r, csl] = ex * si
        return carry

    lax.fori_loop(0, ROWS, row_body, 0)

    def fire_body(r, carry):
        pltpu.async_copy(ex_v.at[r], acc_d.at[dst2_v.at[r]], sem_d, add=True)
        pltpu.async_copy(exsi_v.at[r], acc_n.at[dst2_v.at[r]], sem_n,
                         add=True)
        return carry

    lax.fori_loop(0, ROWS, fire_body, 0)
    pltpu.sync_copy(exf_v, ex_hbm.at[pl.ds(base, EW)])

    def drain_body(r, carry):
        pltpu.make_async_copy(ex_v.at[0], acc_d.at[dst2_v.at[0]],
                              sem_d).wait()
        pltpu.make_async_copy(exsi_v.at[0], acc_n.at[dst2_v.at[0]],
                              sem_n).wait()
        return carry

    lax.fori_loop(0, ROWS, drain_body, 0)

    plsc.subcore_barrier()

    @pl.when(s == 0)
    def _flush():
        pltpu.sync_copy(acc_d, pd_hbm.at[pl.ds(c * NP, NP)])
        pltpu.sync_copy(acc_n, pn_hbm.at[pl.ds(c * NP, NP)])


@functools.partial(
    pl.kernel,
    out_type=[
        jax.ShapeDtypeStruct((N_NODES,), jnp.float32),
        jax.ShapeDtypeStruct((N_EDGES,), jnp.float32),
    ],
    mesh=_MESH,
    compiler_params=pltpu.CompilerParams(needs_layout_passes=False),
    scratch_types=[
        pltpu.VMEM((NP,), jnp.float32),
        pltpu.VMEM((NP,), jnp.float32),
        pltpu.VMEM((EW,), jnp.float32),
        pltpu.VMEM((EW,), jnp.int32),
        pltpu.VMEM((EW,), jnp.float32),
        pltpu.VMEM((NODE_CHUNK,), jnp.float32),
        pltpu.VMEM((NODE_CHUNK,), jnp.float32),
        pltpu.VMEM((NODE_CHUNK,), jnp.float32),
    ],
)
def _sc_norm_kernel(pd_hbm, pn_hbm, ex_hbm, dst_hbm,
                    out_hbm, attn_hbm,
                    d0_v, d1_v, ex_v, dst_v, at_v, n0_v, n1_v, o_v):
    c = lax.axis_index("c")
    s = lax.axis_index("s")
    wid = s * NC + c

    base = wid * EW
    pltpu.sync_copy(pd_hbm.at[pl.ds(0, NP)], d0_v)
    pltpu.sync_copy(pd_hbm.at[pl.ds(NP, NP)], d1_v)
    pltpu.sync_copy(ex_hbm.at[pl.ds(base, EW)], ex_v)
    pltpu.sync_copy(dst_hbm.at[pl.ds(base, EW)], dst_v)
    nb = wid * NODE_CHUNK
    pltpu.sync_copy(pn_hbm.at[pl.ds(nb, NODE_CHUNK)], n0_v)
    pltpu.sync_copy(pn_hbm.at[pl.ds(NP + nb, NODE_CHUNK)], n1_v)

    def recip_body(i, carry):
        sl = pl.ds(i * 16, 16)
        d0_v[sl] = 1.0 / (d0_v[sl] + d1_v[sl] + 1e-16)
        return carry

    lax.fori_loop(0, NP // 16, recip_body, 0)

    def out_body(i, carry):
        sl = pl.ds(i * 16, 16)
        o_v[sl] = (n0_v[sl] + n1_v[sl]) * d0_v[pl.ds(nb + i * 16, 16)]
        return carry

    lax.fori_loop(0, NODE_CHUNK // 16, out_body, 0)

    @pl.when(wid < NW - 1)
    def _full_chunk():
        pltpu.sync_copy(o_v, out_hbm.at[pl.ds(nb, NODE_CHUNK)])

    @pl.when(wid == NW - 1)
    def _tail_chunk():
        pltpu.sync_copy(o_v.at[pl.ds(0, TAIL)],
                        out_hbm.at[pl.ds((NW - 1) * NODE_CHUNK, TAIL)])

    def att_body(r, carry):
        for j in range(GROUPS):
            sl = pl.ds(r * COLS + j * 16, 16)
            rinv = plsc.load_gather(d0_v, [dst_v[sl]])
            at_v[sl] = ex_v[sl] * rinv
        return carry

    lax.fori_loop(0, ROWS, att_body, 0)
    pltpu.sync_copy(at_v, attn_hbm.at[pl.ds(base, EW)])


def kernel(x, edge_index, edge_attr, W, node_W, edge_W):
    Wblk = jnp.kron(jnp.eye(8, dtype=jnp.float32), edge_W)
    Cdiff = jnp.kron(jnp.eye(8, dtype=jnp.float32),
                     jnp.array([[1.0], [-1.0]], dtype=jnp.float32))
    attr8 = edge_attr.reshape(N_EDGES // 8, 128)
    s1d, pair2, g02 = _tc_fused(x, W.reshape(1, 128), attr8, Wblk, Cdiff)

    src = edge_index[0]
    dst = edge_index[1]
    nw_flat = jnp.concatenate(
        [node_W.reshape(4), jnp.zeros((12,), jnp.float32)])

    ex1, pd, pn = _sc_edge_kernel(
        s1d, src, dst, g02.reshape(N_EDGES), nw_flat)
    out, attn = _sc_norm_kernel(pd, pn, ex1, dst)

    return (out, attn, pair2.reshape(N_EDGES, 2))


# speedup vs baseline: 1.0927x; 1.0927x over previous
import functools

import jax
import jax.numpy as jnp
from jax import lax
from jax.experimental import pallas as pl
from jax.experimental.pallas import tpu as pltpu
from jax.experimental.pallas import tpu_sc as plsc

N_NODES = 10000
N_EDGES = 320000
NP = 10240
NC, NS = 2, 16
EW = N_EDGES // NS
COLS = 80
ROWS = EW // COLS
GROUPS = COLS // 16
NODE_CHUNK = NP // NS
TAIL = N_NODES - (NS - 1) * NODE_CHUNK

_MESH = plsc.VectorSubcoreMesh(
    core_axis_name="c", subcore_axis_name="s", num_cores=NC, num_subcores=NS
)


def _tc_fused(x, Wt, attr8, Wblk, Cdiff):
    E8 = N_EDGES // 8
    EB = E8 // 10

    def body(x_ref, w_ref, a_ref, wb_ref, c_ref, s_ref, p_ref, g_ref):
        @pl.when(pl.program_id(0) == 0)
        def _node_scores():
            s_ref[...] = jnp.sum(x_ref[...] * w_ref[...], axis=1)

        p2 = jnp.dot(a_ref[...], wb_ref[...],
                     preferred_element_type=jnp.float32)
        p_ref[...] = p2
        d = jnp.dot(p2, c_ref[...], preferred_element_type=jnp.float32)
        g_ref[...] = 1.0 / (1.0 + jnp.exp(-d))

    return pl.pallas_call(
        body,
        grid=(10,),
        in_specs=[
            pl.BlockSpec((N_NODES, 128), lambda i: (0, 0)),
            pl.BlockSpec((1, 128), lambda i: (0, 0)),
            pl.BlockSpec((EB, 128), lambda i: (i, 0)),
            pl.BlockSpec((128, 16), lambda i: (0, 0)),
            pl.BlockSpec((16, 8), lambda i: (0, 0)),
        ],
        out_specs=[
            pl.BlockSpec((N_NODES,), lambda i: (0,)),
            pl.BlockSpec((EB, 16), lambda i: (i, 0)),
            pl.BlockSpec((EB, 8), lambda i: (i, 0)),
        ],
        out_shape=[
            jax.ShapeDtypeStruct((N_NODES,), jnp.float32),
            jax.ShapeDtypeStruct((E8, 16), jnp.float32),
            jax.ShapeDtypeStruct((E8, 8), jnp.float32),
        ],
    )(x, Wt, attr8, Wblk, Cdiff)


@functools.partial(
    pl.kernel,
    out_type=[
        jax.ShapeDtypeStruct((N_NODES,), jnp.float32),
        jax.ShapeDtypeStruct((N_EDGES,), jnp.float32),
    ],
    mesh=_MESH,
    compiler_params=pltpu.CompilerParams(needs_layout_passes=False),
    scratch_types=[
        pltpu.VMEM((N_NODES,), jnp.float32),
        pltpu.VMEM((EW,), jnp.int32),
        pltpu.VMEM((EW,), jnp.int32),
        pltpu.VMEM((EW,), jnp.float32),
        pltpu.VMEM((16,), jnp.float32),
        pltpu.VMEM((EW,), jnp.float32),
        pltpu.VMEM((EW,), jnp.float32),
        pltpu.VMEM((NP,), jnp.float32),
        pltpu.VMEM((NODE_CHUNK,), jnp.float32),
        pltpu.VMEM((NODE_CHUNK,), jnp.float32),
        pltpu.VMEM_SHARED((NP,), jnp.float32),
        pltpu.VMEM_SHARED((NP,), jnp.float32),
        pltpu.SemaphoreType.DMA,
        pltpu.SemaphoreType.DMA,
    ],
)
def _sc_attention_kernel(s_hbm, src_hbm, dst_hbm, g0_hbm, nw_hbm,
                         out_hbm, attn_hbm,
                         s_v, src_v, dst_v, g0_v, nw_v, ex_v, exsi_v,
                         r_v, n_v, o_v, acc_d, acc_n, sem_d, sem_n):
    c = lax.axis_index("c")
    s = lax.axis_index("s")

    @pl.when(c == 0)
    def _core0():
        def zero_body(i, carry):
            o_v[pl.ds(i * 16, 16)] = jnp.zeros((16,), jnp.float32)
            return carry

        lax.fori_loop(0, NODE_CHUNK // 16, zero_body, 0)
        pltpu.sync_copy(o_v, acc_d.at[pl.ds(s * NODE_CHUNK, NODE_CHUNK)])
        pltpu.sync_copy(o_v, acc_n.at[pl.ds(s * NODE_CHUNK, NODE_CHUNK)])

        base = s * EW
        pltpu.sync_copy(s_hbm, s_v)
        pltpu.sync_copy(src_hbm.at[pl.ds(base, EW)], src_v)
        pltpu.sync_copy(dst_hbm.at[pl.ds(base, EW)], dst_v)
        pltpu.sync_copy(g0_hbm.at[pl.ds(base, EW)], g0_v)
        pltpu.sync_copy(nw_hbm, nw_v)

        plsc.subcore_barrier()

        nw = nw_v[pl.ds(0, 16)]
        w00 = nw[0]
        w01 = nw[1]
        w10 = nw[2]
        w11 = nw[3]

        def row_body(r, carry):
            for j in range(GROUPS):
                sl = pl.ds(r * COLS + j * 16, 16)
                src16 = src_v[sl]
                dst16 = dst_v[sl]
                g0 = g0_v[sl]
                si = plsc.load_gather(s_v, [src16])
                sj = plsc.load_gather(s_v, [dst16])
                a0 = si * w00 + sj * w10
                a1 = si * w01 + sj * w11
                a0 = jnp.where(a0 >= 0.0, a0, a0 * 0.2)
                a1 = jnp.where(a1 >= 0.0, a1, a1 * 0.2)
                logit = a0 * g0 + a1 * (1.0 - g0)
                ex = jnp.exp(logit)
                ex_v[sl] = ex
                exsi_v[sl] = ex * si
            return carry

        lax.fori_loop(0, ROWS, row_body, 0)

        def fire_body(r, carry):
            rsl = pl.ds(r * COLS, COLS)
            pltpu.async_copy(ex_v.at[rsl], acc_d.at[dst_v.at[rsl]], sem_d,
                             add=True)
            pltpu.async_copy(exsi_v.at[rsl], acc_n.at[dst_v.at[rsl]], sem_n,
                             add=True)
            return carry

        lax.fori_loop(0, ROWS, fire_body, 0)

        def drain_body(r, carry):
            rsl = pl.ds(0, COLS)
            pltpu.make_async_copy(ex_v.at[rsl], acc_d.at[dst_v.at[rsl]],
                                  sem_d).wait()
            pltpu.make_async_copy(exsi_v.at[rsl], acc_n.at[dst_v.at[rsl]],
                                  sem_n).wait()
            return carry

        lax.fori_loop(0, ROWS, drain_body, 0)

        plsc.subcore_barrier()

        pltpu.sync_copy(acc_d, r_v)
        nb = s * NODE_CHUNK
        pltpu.sync_copy(acc_n.at[pl.ds(nb, NODE_CHUNK)], n_v)

        def recip_body(i, carry):
            sl = pl.ds(i * 16, 16)
            r_v[sl] = 1.0 / (r_v[sl] + 1e-16)
            return carry

        lax.fori_loop(0, NP // 16, recip_body, 0)

        def out_body(i, carry):
            sl = pl.ds(i * 16, 16)
            o_v[sl] = n_v[sl] * r_v[pl.ds(nb + i * 16, 16)]
            return carry

        lax.fori_loop(0, NODE_CHUNK // 16, out_body, 0)

        @pl.when(s < NS - 1)
        def _full_chunk():
            pltpu.sync_copy(o_v, out_hbm.at[pl.ds(nb, NODE_CHUNK)])

        @pl.when(s == NS - 1)
        def _tail_chunk():
            pltpu.sync_copy(o_v.at[pl.ds(0, TAIL)],
                            out_hbm.at[pl.ds((NS - 1) * NODE_CHUNK, TAIL)])

        def att_body(r, carry):
            for j in range(GROUPS):
                sl = pl.ds(r * COLS + j * 16, 16)
                rinv = plsc.load_gather(r_v, [dst_v[sl]])
                exsi_v[sl] = ex_v[sl] * rinv
            return carry

        lax.fori_loop(0, ROWS, att_body, 0)
        pltpu.sync_copy(exsi_v, attn_hbm.at[pl.ds(base, EW)])


def kernel(x, edge_index, edge_attr, W, node_W, edge_W):
    Wblk = jnp.kron(jnp.eye(8, dtype=jnp.float32), edge_W)
    Cdiff = jnp.kron(jnp.eye(8, dtype=jnp.float32),
                     jnp.array([[1.0], [-1.0]], dtype=jnp.float32))
    attr8 = edge_attr.reshape(N_EDGES // 8, 128)
    s1d, pair2, g02 = _tc_fused(x, W.reshape(1, 128), attr8, Wblk, Cdiff)

    src = edge_index[0]
    dst = edge_index[1]
    nw_flat = jnp.concatenate(
        [node_W.reshape(4), jnp.zeros((12,), jnp.float32)])

    out, attn = _sc_attention_kernel(
        s1d, src, dst, g02.reshape(N_EDGES), nw_flat)

    return (out, attn, pair2.reshape(N_EDGES, 2))
